# Initial kernel scaffold; baseline (speedup 1.0000x reference)
import jax
import jax.numpy as jnp
from jax.experimental import pallas as pl


def kernel(input_ids, token_table, position_table, ctx):
    raise NotImplementedError("write your pallas kernel here")


# speedup vs baseline: 3.4657x; 3.4657x over previous
import functools

import jax
import jax.numpy as jnp
from jax import lax
from jax.experimental import pallas as pl
from jax.experimental.pallas import tpu as pltpu
from jax.experimental.pallas import tpu_sc as plsc

VOCAB = 100000
EMBED_DIM = 128
N_CTX = 16
B = 1024
L = 200

_NC = 2
_NS = 16
_NW = _NC * _NS
_BPW = B // _NW

_G0_OFF, _G0_LEN = 16, 96
_G1_OFF, _G1_LEN = 112, 88


def _sc_embed(ids_hbm, base_hbm, tok_hbm, out_hbm, idx_v, work_v, sem):
  wid = lax.axis_index("s") * _NC + lax.axis_index("c")
  base_b = wid * _BPW

  def body(i, carry):
    pltpu.sync_copy(ids_hbm.at[base_b + i, :], idx_v)

    @pl.when(i == 0)
    def _():
      pltpu.sync_copy(base_hbm, work_v)

    @pl.when(i != 0)
    def _():
      pltpu.sync_copy(base_hbm.at[pl.ds(N_CTX, L - N_CTX)],
                      work_v.at[pl.ds(N_CTX, L - N_CTX)])

    cp0 = pltpu.async_copy(
        tok_hbm.at[idx_v.at[pl.ds(_G0_OFF, _G0_LEN)]],
        work_v.at[pl.ds(_G0_OFF, _G0_LEN)], sem, add=True)
    cp1 = pltpu.async_copy(
        tok_hbm.at[idx_v.at[pl.ds(_G1_OFF, _G1_LEN)]],
        work_v.at[pl.ds(_G1_OFF, _G1_LEN)], sem, add=True)
    cp0.wait()
    cp1.wait()

    pltpu.sync_copy(work_v, out_hbm.at[base_b + i])
    return carry

  lax.fori_loop(0, _BPW, body, 0)


@jax.jit
def kernel(input_ids, token_table, position_table, ctx):
  ids = input_ids.astype(jnp.int32)
  prefix = ctx[:N_CTX] + position_table[:N_CTX]
  base = jnp.concatenate([prefix, position_table[N_CTX:L]], axis=0)

  mesh = plsc.VectorSubcoreMesh(core_axis_name="c", subcore_axis_name="s")
  run = pl.kernel(
      _sc_embed,
      out_type=jax.ShapeDtypeStruct((B, L, EMBED_DIM), jnp.float32),
      mesh=mesh,
      scratch_types=[
          pltpu.VMEM((L,), jnp.int32),
          pltpu.VMEM((L, EMBED_DIM), jnp.float32),
          pltpu.SemaphoreType.DMA,
      ],
  )
  return run(ids, base, token_table)
